# baseline (speedup 1.0000x reference)
import jax
import jax.numpy as jnp
from jax.experimental import pallas as pl
from jax.experimental.pallas import tpu as pltpu

MAG = 0.5

_N, _M = 16384, 2048
_CH_ROWS = 512
_NCH = _N // _CH_ROWS
_DEPTH = 4


def _map_block(c):
    stp = jnp.where(c == 1.0, jnp.float32(MAG), jnp.float32(1.0))
    return jnp.where(c == -1.0, jnp.float32(1.0 / MAG), stp)


def _stream_body(cond_hbm, out_hbm, *rest):
    inb = rest[0:_DEPTH]
    oub = rest[_DEPTH:2 * _DEPTH]
    sin = rest[2 * _DEPTH:3 * _DEPTH]
    sout = rest[3 * _DEPTH:4 * _DEPTH]

    for j in range(_DEPTH):
        pltpu.async_copy(
            cond_hbm.at[pl.ds(j * _CH_ROWS, _CH_ROWS), :], inb[j], sin[j])

    @pl.loop(0, _NCH // _DEPTH)
    def _outer(o):
        base = o * _DEPTH
        for j in range(_DEPTH):
            g = base + j
            pltpu.make_async_copy(
                cond_hbm.at[pl.ds(0, _CH_ROWS), :], inb[j], sin[j]).wait()

            @pl.when(g >= _DEPTH)
            def _():
                pltpu.make_async_copy(
                    oub[j], out_hbm.at[pl.ds(0, _CH_ROWS), :], sout[j]).wait()

            oub[j][...] = _map_block(inb[j][...])
            pltpu.async_copy(
                oub[j], out_hbm.at[pl.ds(g * _CH_ROWS, _CH_ROWS), :], sout[j])

            @pl.when(g + _DEPTH < _NCH)
            def _():
                pltpu.async_copy(
                    cond_hbm.at[pl.ds((g + _DEPTH) * _CH_ROWS, _CH_ROWS), :],
                    inb[j], sin[j])

    for j in range(_DEPTH):
        pltpu.make_async_copy(
            oub[j], out_hbm.at[pl.ds(0, _CH_ROWS), :], sout[j]).wait()


def kernel(cond):
    n, m = cond.shape
    return pl.pallas_call(
        _stream_body,
        in_specs=[pl.BlockSpec(memory_space=pltpu.HBM)],
        out_specs=pl.BlockSpec(memory_space=pltpu.HBM),
        out_shape=jax.ShapeDtypeStruct((n, m), cond.dtype),
        scratch_shapes=(
            [pltpu.VMEM((_CH_ROWS, _M), jnp.float32) for _ in range(2 * _DEPTH)]
            + [pltpu.SemaphoreType.DMA for _ in range(2 * _DEPTH)]
        ),
    )(cond)
